# baseline (speedup 1.0000x reference)
import functools

import jax
import jax.numpy as jnp
from jax import lax
from jax.experimental import pallas as pl
from jax.experimental.pallas import tpu as pltpu
from jax.experimental.pallas import tpu_sc as plsc

NUM_NODES = 100000
HIDDEN_DIM = 128

NUM_WORKERS = 32
CHUNK = 128
CHUNKS_PER_W = 25
B_PER_W = CHUNK * CHUNKS_PER_W
B_PAD = NUM_WORKERS * B_PER_W

_mesh = plsc.VectorSubcoreMesh(core_axis_name="c", subcore_axis_name="s")


@functools.partial(
    pl.kernel,
    mesh=_mesh,
    out_type=jax.ShapeDtypeStruct((B_PAD, HIDDEN_DIM), jnp.float32),
    scratch_types=[
        pltpu.VMEM((B_PER_W,), jnp.int32),
        pltpu.VMEM((2, CHUNK, HIDDEN_DIM), jnp.float32),
        pltpu.SemaphoreType.DMA,
        pltpu.SemaphoreType.DMA,
    ],
)
def _gather_kernel(idx_hbm, table_hbm, out_hbm, idx_v, rows_v, gsem, osem):
    wid = lax.axis_index("s") * 2 + lax.axis_index("c")
    base = wid * B_PER_W
    pltpu.sync_copy(idx_hbm.at[pl.ds(base, B_PER_W)], idx_v)

    def body(j, _):
        off = j * CHUNK
        pltpu.async_copy(
            table_hbm.at[idx_v.at[pl.ds(off, CHUNK)]], rows_v.at[0], gsem
        ).wait()
        pltpu.async_copy(
            rows_v.at[0], out_hbm.at[pl.ds(base + off, CHUNK)], osem
        ).wait()
        return 0

    lax.fori_loop(0, CHUNKS_PER_W, body, 0)


def kernel(node_ids, table):
    node_ids = node_ids.reshape(-1).astype(jnp.int32)
    idx_padded = jnp.concatenate(
        [node_ids, jnp.zeros((B_PAD - NUM_NODES,), jnp.int32)]
    )
    out = _gather_kernel(idx_padded, table)
    return out[:NUM_NODES]


# speedup vs baseline: 1.1626x; 1.1626x over previous
import functools

import jax
import jax.numpy as jnp
from jax import lax
from jax.experimental import pallas as pl
from jax.experimental.pallas import tpu as pltpu
from jax.experimental.pallas import tpu_sc as plsc

NUM_NODES = 100000
HIDDEN_DIM = 128

NUM_WORKERS = 32
CHUNK = 128
CHUNKS_PER_W = 25
B_PER_W = CHUNK * CHUNKS_PER_W
B_PAD = NUM_WORKERS * B_PER_W

NBUF = 4

_mesh = plsc.VectorSubcoreMesh(core_axis_name="c", subcore_axis_name="s")


@functools.partial(
    pl.kernel,
    mesh=_mesh,
    out_type=jax.ShapeDtypeStruct((B_PAD, HIDDEN_DIM), jnp.float32),
    scratch_types=[
        pltpu.VMEM((B_PER_W,), jnp.int32),
        pltpu.VMEM((NBUF, CHUNK, HIDDEN_DIM), jnp.float32),
        pltpu.SemaphoreType.DMA,
        pltpu.SemaphoreType.DMA,
    ],
)
def _gather_kernel(idx_hbm, table_hbm, out_hbm, idx_v, rows_v, gsem, osem):
    wid = lax.axis_index("s") * 2 + lax.axis_index("c")
    base = wid * B_PER_W
    pltpu.sync_copy(idx_hbm.at[pl.ds(base, B_PER_W)], idx_v)

    def start_g(j):
        return pltpu.async_copy(
            table_hbm.at[idx_v.at[pl.ds(j * CHUNK, CHUNK)]],
            rows_v.at[j % NBUF],
            gsem,
        )

    def start_w(j):
        return pltpu.async_copy(
            rows_v.at[j % NBUF],
            out_hbm.at[pl.ds(base + j * CHUNK, CHUNK)],
            osem,
        )

    g = {}
    w = {}
    for b in range(NBUF - 1):
        g[b] = start_g(b)
    for j in range(CHUNKS_PER_W):
        g[j].wait()
        w[j] = start_w(j)
        if j >= 1:
            w[j - 1].wait()
        if j + NBUF - 1 < CHUNKS_PER_W:
            g[j + NBUF - 1] = start_g(j + NBUF - 1)
    w[CHUNKS_PER_W - 1].wait()


def kernel(node_ids, table):
    node_ids = node_ids.reshape(-1).astype(jnp.int32)
    idx_padded = jnp.concatenate(
        [node_ids, jnp.zeros((B_PAD - NUM_NODES,), jnp.int32)]
    )
    out = _gather_kernel(idx_padded, table)
    return out[:NUM_NODES]


# speedup vs baseline: 3.5112x; 3.0203x over previous
import functools

import jax
import jax.numpy as jnp
from jax import lax
from jax.experimental import pallas as pl
from jax.experimental.pallas import tpu as pltpu
from jax.experimental.pallas import tpu_sc as plsc

NUM_NODES = 100000
HIDDEN_DIM = 128

CHUNK = 128
NBUF = 4
NS = 16

N_C0 = 5120
N_C1 = 1136
N_LAST = NUM_NODES - NS * N_C0 - (NS - 1) * N_C1
assert N_LAST % 8 == 0 and N_LAST > 0

_mesh = plsc.VectorSubcoreMesh(core_axis_name="c", subcore_axis_name="s")


def _pipeline(idx_hbm, table_hbm, out_hbm, idx_v, rows_v, gsem, osem,
              base, count):
    nfull = count // CHUNK
    tail = count % CHUNK
    ngr = nfull // NBUF
    assert nfull % NBUF == 0 and ngr >= 1

    pltpu.sync_copy(idx_hbm.at[pl.ds(base, count)],
                    idx_v.at[pl.ds(0, count)])

    def g_desc(j, b, size=CHUNK):
        dst = rows_v.at[b] if size == CHUNK else rows_v.at[b, pl.ds(0, size)]
        return pltpu.make_async_copy(
            table_hbm.at[idx_v.at[pl.ds(j * CHUNK, size)]], dst, gsem)

    def w_desc(j, b, size=CHUNK):
        src = rows_v.at[b] if size == CHUNK else rows_v.at[b, pl.ds(0, size)]
        return pltpu.make_async_copy(
            src, out_hbm.at[pl.ds(base + j * CHUNK, size)], osem)

    for b in range(NBUF):
        g_desc(b, b).start()

    def grp_body(g, _):
        cb = g * NBUF
        for b in range(NBUF):
            g_desc(cb + b, b).wait()
            w_desc(cb + b, b).start()
        for b in range(NBUF):
            w_desc(cb + b, b).wait()
            g_desc(cb + NBUF + b, b).start()
        return 0

    lax.fori_loop(0, ngr - 1, grp_body, 0)

    cb = (ngr - 1) * NBUF
    for b in range(NBUF):
        g_desc(cb + b, b).wait()
        w_desc(cb + b, b).start()
    if tail:
        w_desc(cb, 0).wait()
        g_desc(nfull, 0, tail).start()
        g_desc(nfull, 0, tail).wait()
        w_desc(nfull, 0, tail).start()
        w_desc(nfull, 0, tail).wait()
        for b in range(1, NBUF):
            w_desc(cb + b, b).wait()
    else:
        for b in range(NBUF):
            w_desc(cb + b, b).wait()


@functools.partial(
    pl.kernel,
    mesh=_mesh,
    out_type=jax.ShapeDtypeStruct((NUM_NODES, HIDDEN_DIM), jnp.float32),
    scratch_types=[
        pltpu.VMEM((N_C0,), jnp.int32),
        pltpu.VMEM((NBUF, CHUNK, HIDDEN_DIM), jnp.float32),
        pltpu.SemaphoreType.DMA,
        pltpu.SemaphoreType.DMA,
    ],
)
def _gather_kernel(idx_hbm, table_hbm, out_hbm, idx_v, rows_v, gsem, osem):
    c = lax.axis_index("c")
    s = lax.axis_index("s")
    args = (idx_hbm, table_hbm, out_hbm, idx_v, rows_v, gsem, osem)

    @pl.when(c == 0)
    def _():
        _pipeline(*args, base=s * N_C0, count=N_C0)

    @pl.when((c == 1) & (s < NS - 1))
    def _():
        _pipeline(*args, base=NS * N_C0 + s * N_C1, count=N_C1)

    @pl.when((c == 1) & (s == NS - 1))
    def _():
        _pipeline(*args, base=NS * N_C0 + (NS - 1) * N_C1, count=N_LAST)


def kernel(node_ids, table):
    node_ids = node_ids.reshape(-1).astype(jnp.int32)
    return _gather_kernel(node_ids, table)


# speedup vs baseline: 3.7276x; 1.0616x over previous
import functools

import jax
import jax.numpy as jnp
from jax import lax
from jax.experimental import pallas as pl
from jax.experimental.pallas import tpu as pltpu
from jax.experimental.pallas import tpu_sc as plsc

NUM_NODES = 100000
HIDDEN_DIM = 128

CHUNK = 128
NBUF = 4
NS = 16

N_C0 = 3856
N_C1 = 2392
N_LAST = NUM_NODES - NS * N_C0 - (NS - 1) * N_C1
assert N_LAST % 8 == 0 and N_LAST > 0

_mesh = plsc.VectorSubcoreMesh(core_axis_name="c", subcore_axis_name="s")


def _pipeline(idx_hbm, table_hbm, out_hbm, idx_v, rows_v, gsem, osem,
              base, count):
    nfull = count // CHUNK
    tail = count % CHUNK
    ngr = nfull // NBUF
    rem = nfull - ngr * NBUF
    assert ngr >= 1

    pltpu.sync_copy(idx_hbm.at[pl.ds(base, count)],
                    idx_v.at[pl.ds(0, count)])

    def g_desc(j, b, size=CHUNK):
        dst = rows_v.at[b] if size == CHUNK else rows_v.at[b, pl.ds(0, size)]
        return pltpu.make_async_copy(
            table_hbm.at[idx_v.at[pl.ds(j * CHUNK, size)]], dst, gsem)

    def w_desc(j, b, size=CHUNK):
        src = rows_v.at[b] if size == CHUNK else rows_v.at[b, pl.ds(0, size)]
        return pltpu.make_async_copy(
            src, out_hbm.at[pl.ds(base + j * CHUNK, size)], osem)

    for b in range(NBUF):
        g_desc(b, b).start()

    def grp_body(g, _):
        cb = g * NBUF
        for b in range(NBUF):
            g_desc(cb + b, b).wait()
            w_desc(cb + b, b).start()
        for b in range(NBUF):
            w_desc(cb + b, b).wait()
            g_desc(cb + NBUF + b, b).start()
        return 0

    lax.fori_loop(0, ngr - 1, grp_body, 0)

    extras = [(ngr * NBUF + k, CHUNK) for k in range(rem)]
    if tail:
        extras.append((nfull, tail))
    cb = (ngr - 1) * NBUF
    for b in range(NBUF):
        g_desc(cb + b, b).wait()
        w_desc(cb + b, b).start()
    for k, (j, size) in enumerate(extras):
        w_desc(cb + k, k).wait()
        g_desc(j, k, size).start()
    for b in range(len(extras), NBUF):
        w_desc(cb + b, b).wait()
    for k, (j, size) in enumerate(extras):
        g_desc(j, k, size).wait()
        w_desc(j, k, size).start()
    for k, (j, size) in enumerate(extras):
        w_desc(j, k, size).wait()


@functools.partial(
    pl.kernel,
    mesh=_mesh,
    out_type=jax.ShapeDtypeStruct((NUM_NODES, HIDDEN_DIM), jnp.float32),
    scratch_types=[
        pltpu.VMEM((N_C0,), jnp.int32),
        pltpu.VMEM((NBUF, CHUNK, HIDDEN_DIM), jnp.float32),
        pltpu.SemaphoreType.DMA,
        pltpu.SemaphoreType.DMA,
    ],
)
def _gather_kernel(idx_hbm, table_hbm, out_hbm, idx_v, rows_v, gsem, osem):
    c = lax.axis_index("c")
    s = lax.axis_index("s")
    args = (idx_hbm, table_hbm, out_hbm, idx_v, rows_v, gsem, osem)

    @pl.when(c == 0)
    def _():
        _pipeline(*args, base=s * N_C0, count=N_C0)

    @pl.when((c == 1) & (s < NS - 1))
    def _():
        _pipeline(*args, base=NS * N_C0 + s * N_C1, count=N_C1)

    @pl.when((c == 1) & (s == NS - 1))
    def _():
        _pipeline(*args, base=NS * N_C0 + (NS - 1) * N_C1, count=N_LAST)


def kernel(node_ids, table):
    node_ids = node_ids.reshape(-1).astype(jnp.int32)
    return _gather_kernel(node_ids, table)
